# baseline (speedup 1.0000x reference)
import functools

import jax
import jax.numpy as jnp
from jax import lax
from jax.experimental import pallas as pl
from jax.experimental.pallas import tpu as pltpu
from jax.experimental.pallas import tpu_sc as plsc

N_SEEDS = 1_000_000
PLANE = 2097152
BLK = 128
N_BLKS = (N_SEEDS + BLK - 1) // BLK
OUT_WORDS = N_BLKS * 512
CHUNK = 2048
BPC = CHUNK // BLK
W_CHUNK = 3 * CHUNK
N_CHUNKS = (N_BLKS + BPC - 1) // BPC
TAIL_K = N_CHUNKS - 1
TAIL_SEEDS = N_SEEDS - TAIL_K * CHUNK
NC, NS = 2, 16
NW = NC * NS
GROUPS = CHUNK // 16
T_STEPS = (N_CHUNKS + NW - 1) // NW

_mesh = plsc.VectorSubcoreMesh(core_axis_name="c", subcore_axis_name="s",
                               num_cores=NC, num_subcores=NS)


@functools.partial(
    pl.kernel,
    out_type=jax.ShapeDtypeStruct((OUT_WORDS,), jnp.float32),
    mesh=_mesh,
    scratch_types=[
        pltpu.VMEM((W_CHUNK,), jnp.int32),
        pltpu.VMEM((W_CHUNK,), jnp.int32),
        pltpu.VMEM((W_CHUNK,), jnp.int32),
        pltpu.VMEM((W_CHUNK,), jnp.int32),
        pltpu.VMEM((W_CHUNK,), jnp.int32),
        pltpu.VMEM((W_CHUNK,), jnp.int32),
        pltpu.VMEM((W_CHUNK,), jnp.float32),
        pltpu.VMEM((W_CHUNK,), jnp.float32),
        pltpu.VMEM((W_CHUNK,), jnp.float32),
        pltpu.SemaphoreType.DMA,
        pltpu.SemaphoreType.DMA,
        pltpu.SemaphoreType.DMA,
        pltpu.SemaphoreType.DMA,
        pltpu.SemaphoreType.DMA,
        pltpu.SemaphoreType.DMA,
    ],
    compiler_params=pltpu.CompilerParams(needs_layout_passes=False),
)
def _gather(seeds_hbm, table_hbm, out_hbm,
            sv_a, sv_b, sv_c, idx_a, idx_b, idx_c, dest_a, dest_b, dest_c,
            sem_ga, sem_gb, sem_gc, sem_oa, sem_ob, sem_oc):
    wid = lax.axis_index("s") * NC + lax.axis_index("c")
    bufs = [(sv_a, idx_a, dest_a, sem_oa, sem_ga),
            (sv_b, idx_b, dest_b, sem_ob, sem_gb),
            (sv_c, idx_c, dest_c, sem_oc, sem_gc)]
    zeros16 = jnp.zeros((16,), jnp.int32)

    def load_and_index(k, sv, idxv):
        base = k * CHUNK

        @pl.when(k < TAIL_K)
        def _():
            for c in range(3):
                pltpu.sync_copy(seeds_hbm.at[pl.ds(c * N_SEEDS + base, CHUNK)],
                                sv.at[pl.ds(c * CHUNK, CHUNK)])

        @pl.when(k == TAIL_K)
        def _():
            def zbody(i, _):
                sv[pl.ds(i * 16, 16)] = zeros16
                return 0

            lax.fori_loop(0, W_CHUNK // 16, zbody, 0)
            for c in range(3):
                pltpu.sync_copy(
                    seeds_hbm.at[pl.ds(c * N_SEEDS + base, TAIL_SEEDS)],
                    sv.at[pl.ds(c * CHUNK, TAIL_SEEDS)])

        def group_body(it, _):
            s = it * 16
            bb = it >> 3
            di = bb * 384 + (it & 7) * 16
            x = sv[pl.ds(s, 16)]
            y = sv[pl.ds(CHUNK + s, 16)]
            z = sv[pl.ds(2 * CHUNK + s, 16)]
            flat = (x << 14) | (y << 7) | z
            idxv[pl.ds(di, 16)] = flat
            idxv[pl.ds(di + 128, 16)] = flat + PLANE
            idxv[pl.ds(di + 256, 16)] = flat + 2 * PLANE
            return 0

        lax.fori_loop(0, GROUPS, group_body, 0)

    def issue_outs(k, dest, sem):
        for b in range(BPC):
            blk = k * BPC + b

            @pl.when(blk < N_BLKS)
            def _(b=b, blk=blk):
                pltpu.async_copy(dest.at[pl.ds(b * 384, 384)],
                                 out_hbm.at[pl.ds(blk * 512, 384)], sem)

    def drain_outs(k, dest, sem):
        for b in range(BPC):
            blk = k * BPC + b

            @pl.when(blk < N_BLKS)
            def _(b=b):
                pltpu.make_async_copy(dest.at[pl.ds(b * 384, 384)],
                                      out_hbm.at[pl.ds(0, 384)], sem).wait()

    pending_g = [None, None, None]

    def issue_outs_t(t_done):
        b = t_done % 3
        issue_outs(wid + t_done * NW, bufs[b][2], bufs[b][3])

    def finish_chunk(t_done):
        b = t_done % 3
        pending_g[b].wait()
        issue_outs_t(t_done)

    def drain_outs_t(t_done):
        b = t_done % 3
        drain_outs(wid + t_done * NW, bufs[b][2], bufs[b][3])

    for t in range(T_STEPS):
        b = t % 3
        sv, idxv, dest, sem_o, sem_g = bufs[b]
        k = wid + t * NW

        @pl.when(k < N_CHUNKS)
        def _(t=t, b=b, k=k, sv=sv, idxv=idxv, dest=dest, sem_o=sem_o,
              sem_g=sem_g):
            if t >= 3:
                drain_outs_t(t - 3)
            load_and_index(k, sv, idxv)
            pending_g[b] = pltpu.async_copy(table_hbm.at[idxv], dest, sem_g)
            if t >= 2:
                finish_chunk(t - 2)

    last_k = wid + (T_STEPS - 1) * NW

    @pl.when(last_k < N_CHUNKS)
    def _():
        finish_chunk(T_STEPS - 2)
        finish_chunk(T_STEPS - 1)
        for td in (T_STEPS - 3, T_STEPS - 2, T_STEPS - 1):
            drain_outs_t(td)

    @pl.when(last_k >= N_CHUNKS)
    def _():
        finish_chunk(T_STEPS - 3)
        finish_chunk(T_STEPS - 2)
        for td in (T_STEPS - 4, T_STEPS - 3, T_STEPS - 2):
            drain_outs_t(td)


def kernel(seeds, vector_field):
    seeds_planar = seeds.T.reshape(3 * N_SEEDS)
    table = vector_field.reshape(3 * PLANE)
    out = _gather(seeds_planar, table)
    o3 = out.reshape(N_BLKS, 4, 128).transpose(0, 2, 1).reshape(N_BLKS * 128, 4)
    return o3[:N_SEEDS, :3]


# speedup vs baseline: 1.0481x; 1.0481x over previous
import functools

import jax
import jax.numpy as jnp
from jax import lax
from jax.experimental import pallas as pl
from jax.experimental.pallas import tpu as pltpu
from jax.experimental.pallas import tpu_sc as plsc

N_SEEDS = 1_000_000
PLANE = 2097152
CHUNK = 4000
W_CHUNK = 3 * CHUNK
N_CHUNKS = N_SEEDS // CHUNK
NC, NS = 2, 16
NW = NC * NS
GROUPS = CHUNK // 16
T_STEPS = (N_CHUNKS + NW - 1) // NW

_mesh = plsc.VectorSubcoreMesh(core_axis_name="c", subcore_axis_name="s",
                               num_cores=NC, num_subcores=NS)


@functools.partial(
    pl.kernel,
    out_type=jax.ShapeDtypeStruct((3 * N_SEEDS,), jnp.float32),
    mesh=_mesh,
    scratch_types=[
        pltpu.VMEM((W_CHUNK,), jnp.int32),
        pltpu.VMEM((W_CHUNK,), jnp.int32),
        pltpu.VMEM((W_CHUNK,), jnp.int32),
        pltpu.VMEM((W_CHUNK,), jnp.int32),
        pltpu.VMEM((W_CHUNK,), jnp.int32),
        pltpu.VMEM((W_CHUNK,), jnp.int32),
        pltpu.VMEM((W_CHUNK,), jnp.float32),
        pltpu.VMEM((W_CHUNK,), jnp.float32),
        pltpu.VMEM((W_CHUNK,), jnp.float32),
        pltpu.SemaphoreType.DMA,
        pltpu.SemaphoreType.DMA,
        pltpu.SemaphoreType.DMA,
        pltpu.SemaphoreType.DMA,
        pltpu.SemaphoreType.DMA,
        pltpu.SemaphoreType.DMA,
    ],
    compiler_params=pltpu.CompilerParams(needs_layout_passes=False),
)
def _gather(seeds_hbm, table_hbm, out_hbm,
            sv_a, sv_b, sv_c, idx_a, idx_b, idx_c, dest_a, dest_b, dest_c,
            sem_ga, sem_gb, sem_gc, sem_oa, sem_ob, sem_oc):
    wid = lax.axis_index("s") * NC + lax.axis_index("c")
    bufs = [(sv_a, idx_a, dest_a, sem_oa, sem_ga),
            (sv_b, idx_b, dest_b, sem_ob, sem_gb),
            (sv_c, idx_c, dest_c, sem_oc, sem_gc)]

    def load_and_index(k, sv, idxv):
        base = k * CHUNK
        for c in range(3):
            pltpu.sync_copy(seeds_hbm.at[pl.ds(c * N_SEEDS + base, CHUNK)],
                            sv.at[pl.ds(c * CHUNK, CHUNK)])

        def group_body(g, _):
            s = g * 16
            x = sv[pl.ds(s, 16)]
            y = sv[pl.ds(CHUNK + s, 16)]
            z = sv[pl.ds(2 * CHUNK + s, 16)]
            flat = (x << 14) | (y << 7) | z
            idxv[pl.ds(s, 16)] = flat
            idxv[pl.ds(CHUNK + s, 16)] = flat + PLANE
            idxv[pl.ds(2 * CHUNK + s, 16)] = flat + 2 * PLANE
            return 0

        lax.fori_loop(0, GROUPS, group_body, 0)

    def issue_outs(k, dest, sem):
        base = k * CHUNK
        for c in range(3):
            pltpu.async_copy(dest.at[pl.ds(c * CHUNK, CHUNK)],
                             out_hbm.at[pl.ds(c * N_SEEDS + base, CHUNK)], sem)

    def drain_outs(dest, sem):
        for c in range(3):
            pltpu.make_async_copy(dest.at[pl.ds(c * CHUNK, CHUNK)],
                                  out_hbm.at[pl.ds(0, CHUNK)], sem).wait()

    pending_g = [None, None, None]

    def finish_chunk(t_done):
        b = t_done % 3
        for h in pending_g[b]:
            h.wait()
        issue_outs_t(t_done)

    def issue_outs_t(t_done):
        b = t_done % 3
        issue_outs(wid + t_done * NW, bufs[b][2], bufs[b][3])

    for t in range(T_STEPS):
        b = t % 3
        sv, idxv, dest, sem_o, sem_g = bufs[b]
        k = wid + t * NW

        @pl.when(k < N_CHUNKS)
        def _(t=t, b=b, k=k, sv=sv, idxv=idxv, dest=dest, sem_o=sem_o,
              sem_g=sem_g):
            if t >= 3:
                drain_outs(dest, sem_o)
            load_and_index(k, sv, idxv)
            pending_g[b] = [pltpu.async_copy(table_hbm.at[idxv], dest, sem_g)]
            if t >= 2:
                finish_chunk(t - 2)

    last_k = wid + (T_STEPS - 1) * NW

    @pl.when(last_k < N_CHUNKS)
    def _():
        finish_chunk(T_STEPS - 2)
        finish_chunk(T_STEPS - 1)
        for bb in range(3):
            drain_outs(bufs[bb][2], bufs[bb][3])

    @pl.when(last_k >= N_CHUNKS)
    def _():
        finish_chunk(T_STEPS - 3)
        finish_chunk(T_STEPS - 2)
        for bb in range(3):
            drain_outs(bufs[bb][2], bufs[bb][3])


def kernel(seeds, vector_field):
    seeds_planar = seeds.T.reshape(3 * N_SEEDS)
    table = vector_field.reshape(3 * PLANE)
    out = _gather(seeds_planar, table)
    return out.reshape(3, N_SEEDS).T
